# baseline (speedup 1.0000x reference)
import functools
import jax
import jax.numpy as jnp
from jax import lax
from jax.experimental import pallas as pl
from jax.experimental.pallas import tpu as pltpu
from jax.experimental.pallas import tpu_sc as plsc

N = 10000
E = 320000
D = 128
DP = D // 2
NH = N // 2
L = 16
NC = 2
NS = 16
NW = NC * NS
NPW = 320
NPAD = NW * NPW
C = 2000
NCHUNK = E // C
FT = 10500
MBUF = FT + 2 * C + 64
SRTW = MBUF // 2 + 32
NBIN = N + L
XR = 40
NXC = NH // XR
DUMP = NPW
NEGP = -8421505

_mesh = plsc.VectorSubcoreMesh(
    core_axis_name="c", subcore_axis_name="s", num_cores=NC, num_subcores=NS
)


@functools.partial(
    pl.kernel,
    out_type=jax.ShapeDtypeStruct((NPAD * D,), jnp.float32),
    mesh=_mesh,
    compiler_params=pltpu.CompilerParams(needs_layout_passes=False),
    scratch_types=[
        pltpu.VMEM(((NPW + 1) * D,), jnp.float32),
        pltpu.VMEM(((NPW + 1) * D,), jnp.float32),
        pltpu.VMEM((NPW + 2 * L,), jnp.float32),
        pltpu.VMEM((NBIN,), jnp.int32),
        pltpu.VMEM((MBUF,), jnp.int32),
        pltpu.VMEM((SRTW,), jnp.int32),
        pltpu.VMEM((C,), jnp.int32),
        pltpu.VMEM((C,), jnp.int32),
        pltpu.VMEM((XR, D), jnp.int32),
        pltpu.VMEM((XR, D), jnp.int32),
        pltpu.VMEM((L,), jnp.float32),
        pltpu.SemaphoreType.DMA,
        pltpu.SemaphoreType.DMA,
        pltpu.SemaphoreType.DMA,
        pltpu.SemaphoreType.DMA,
    ],
)
def _mp(z_hbm, ep_hbm, w_hbm, out_hbm,
        s_acc, m_acc, cnt_acc, bins, mpack, srt, chA, chB, xchA, xchB, zv,
        semA, semB, semXA, semXB):
    cid = lax.axis_index("c")
    sid = lax.axis_index("s")
    wid = cid * NS + sid
    lo = wid * NPW

    fzeros = jnp.zeros((L,), jnp.float32)
    fones = jnp.ones((L,), jnp.float32)
    izeros = jnp.zeros((L,), jnp.int32)
    iones = jnp.ones((L,), jnp.int32)
    negs = jnp.full((L,), -3.0e38, jnp.float32)
    dumpv = jnp.full((L,), DUMP << 14, jnp.int32)
    lowm = jnp.full((L,), 16383, jnp.int32)
    highm = jnp.full((L,), -65536, jnp.int32)
    sh16 = jnp.full((L,), 16, jnp.int32)
    lanes = lax.iota(jnp.int32, L)
    e1f = jnp.where(lanes == izeros, fones, fzeros)
    e1i = jnp.where(lanes == izeros, iones, izeros)
    lov14 = jnp.full((L,), lo * 16384, jnp.int32)
    hiv14 = jnp.full((L,), (lo + NPW) * 16384, jnp.int32)

    def init_acc(i, carry):
        s_acc[pl.ds(i * L, L)] = fzeros
        m_acc[pl.ds(i * L, L)] = negs
        return carry
    lax.fori_loop(0, (NPW + 1) * D // L, init_acc, 0)

    def init_cnt(i, carry):
        cnt_acc[pl.ds(i * L, L)] = fzeros
        return carry
    lax.fori_loop(0, (NPW + 2 * L) // L, init_cnt, 0)

    def init_bins(i, carry):
        bins[pl.ds(i * L, L)] = izeros
        return carry
    lax.fori_loop(0, NBIN // L, init_bins, 0)

    def init_srt(i, carry):
        srt[pl.ds(i * L, L)] = izeros
        return carry
    lax.fori_loop(0, SRTW // L, init_srt, 0)

    pltpu.sync_copy(z_hbm, zv)

    def issue_x(k, xb, sem):
        pltpu.async_copy(w_hbm.at[pl.ds(k * XR, XR), :], xb, sem)

    def wait_x(xb, sem):
        pltpu.make_async_copy(w_hbm.at[pl.ds(0, XR), :], xb, sem).wait()

    def walk(xi, xb):
        rbase = xi * (2 * XR)

        def grp(g, carry):
            r0 = rbase + g * L
            sv = bins[pl.ds(r0, L)]
            ev = bins[pl.ds(r0 + 1, L)]
            for t in range(L):
                kvs = sv[t]
                nk = ev[t] - kvs
                slh = g * (L // 2) + (t >> 1)
                hof = (t & 1) * DP

                def edge(ki, carry2):
                    k = kvs + ki
                    sw = srt[pl.ds(lax.shift_right_logical(k, 1), L)][0]
                    ls = lax.shift_right_logical(sw, (k & 1) * L) & 511
                    baseS = ls * D
                    for h in range(DP // L):
                        wv = xb[slh, pl.ds(hof + h * L, L)]
                        lof = plsc.bitcast(lax.shift_left(wv, sh16),
                                           jnp.float32)
                        hif = plsc.bitcast(wv & highm, jnp.float32)
                        plsc.addupdate(
                            s_acc.at[pl.ds(baseS + h * 2 * L, L)], lof)
                        plsc.addupdate(
                            s_acc.at[pl.ds(baseS + h * 2 * L + L, L)], hif)
                        mlo = m_acc[pl.ds(baseS + h * 2 * L, L)]
                        m_acc[pl.ds(baseS + h * 2 * L, L)] = (
                            jnp.maximum(mlo, lof))
                        mhi = m_acc[pl.ds(baseS + h * 2 * L + L, L)]
                        m_acc[pl.ds(baseS + h * 2 * L + L, L)] = (
                            jnp.maximum(mhi, hif))
                    return carry2
                lax.fori_loop(0, nk, edge, 0)
            return carry
        lax.fori_loop(0, 2 * XR // L, grp, 0)

    def flush(m):
        m16 = ((m + (L - 1)) >> 4) << 4
        mal = (m >> 4) << 4

        @pl.when(mal < m16)
        def _():
            v = mpack[pl.ds(mal, L)]
            posv = jnp.full((L,), mal, jnp.int32) + lanes
            mpack[pl.ds(mal, L)] = jnp.where(
                posv >= jnp.full((L,), m, jnp.int32), dumpv, v)

        nv = m16 >> 4

        issue_x(0, xchA, semXA)
        issue_x(1, xchB, semXB)

        def hist(i, carry):
            pv = mpack[pl.ds(i * L, L)]
            for t in range(L):
                dv = pv[t] & 16383
                plsc.addupdate(bins.at[pl.ds(dv, L)], e1i)
            return carry
        lax.fori_loop(0, nv, hist, 0)

        def pfx(i, tot):
            v = bins[pl.ds(i * L, L)]
            cum = plsc.cumsum(v)
            bins[pl.ds(i * L, L)] = cum + jnp.full((L,), tot, jnp.int32)
            return tot + cum[L - 1]
        lax.fori_loop(0, NBIN // L, pfx, 0)

        ne1i = izeros - e1i

        def place(i, carry):
            pv = mpack[pl.ds(i * L, L)]
            for t in range(L):
                lsp = pv[t]
                dv = lsp & 16383
                ls = lax.shift_right_logical(lsp, 14)
                p = bins[pl.ds(dv, L)][0] - 1
                plsc.addupdate(bins.at[pl.ds(dv, L)], ne1i)
                plsc.addupdate(cnt_acc.at[pl.ds(ls, L)], e1f)
                lsh = lax.shift_left(ls, (p & 1) * L)
                plsc.addupdate(
                    srt.at[pl.ds(lax.shift_right_logical(p, 1), L)],
                    jnp.where(lanes == izeros,
                              jnp.full((L,), lsh, jnp.int32), izeros))
            return carry
        lax.fori_loop(0, nv, place, 0)

        def xpair(cp, carry):
            ca = 2 * cp
            wait_x(xchA, semXA)
            walk(ca, xchA)

            @pl.when(ca + 2 < NXC)
            def _():
                issue_x(ca + 2, xchA, semXA)

            @pl.when(ca + 1 < NXC)
            def _():
                wait_x(xchB, semXB)
                walk(ca + 1, xchB)

                @pl.when(ca + 3 < NXC)
                def __():
                    issue_x(ca + 3, xchB, semXB)
            return carry
        lax.fori_loop(0, (NXC + 1) // 2, xpair, 0)

        def rz_bins(i, carry):
            bins[pl.ds(i * L, L)] = izeros
            return carry
        lax.fori_loop(0, NBIN // L, rz_bins, 0)

        def rz_srt(i, carry):
            srt[pl.ds(i * L, L)] = izeros
            return carry
        lax.fori_loop(0, (nv >> 1) + 1, rz_srt, 0)

    def scan_chunk(cb, m0):
        def scan_body(i, off):
            ev = cb[pl.ds(i * L, L)]
            msk = (ev >= lov14) & (ev < hiv14)
            inc = jnp.where(msk, iones, izeros)
            pos = plsc.cumsum(inc)
            idx = jnp.full((L,), off - 1, jnp.int32) + pos
            plsc.store_scatter(mpack, [idx], ev - lov14, mask=msk)
            pc = plsc.all_reduce_population_count(msk)
            return off + pc[0]
        return lax.fori_loop(0, C // L, scan_body, m0)

    issue_chunk = lambda k, cb, sem: pltpu.async_copy(
        ep_hbm.at[pl.ds(k * C, C)], cb, sem)
    wait_chunk = lambda cb, sem: pltpu.make_async_copy(
        ep_hbm.at[pl.ds(0, C)], cb, sem).wait()

    issue_chunk(0, chA, semA)

    def chunk_pair(p, m):
        last = p == NCHUNK // 2

        k0 = 2 * p

        @pl.when(jnp.logical_not(last))
        def _():
            wait_chunk(chA, semA)
            issue_chunk(k0 + 1, chB, semB)
        m = jnp.where(last, m, scan_chunk(chA, m))

        @pl.when(jnp.logical_not(last))
        def _():
            wait_chunk(chB, semB)

            @pl.when(k0 + 2 < NCHUNK)
            def __():
                issue_chunk(k0 + 2, chA, semA)
        m = jnp.where(last, m, scan_chunk(chB, m))
        do_flush = (m >= FT) | (last & (m > 0))

        @pl.when(do_flush)
        def _():
            flush(m)
        return jnp.where(do_flush, 0, m)
    m = lax.fori_loop(0, NCHUNK // 2 + 1, chunk_pair, 0)

    zvec = zv[pl.ds(0, L)]
    z0v = jnp.full((L,), zvec[0])
    z1v = jnp.full((L,), zvec[1])
    z2v = jnp.full((L,), zvec[2])

    def comb_group(ng, carry):
        n0 = ng * L
        cv = cnt_acc[pl.ds(n0, L)]
        scalev = z0v + z1v / jnp.maximum(cv, fones)
        zmxv = jnp.where(cv > fzeros, z2v, fzeros)
        for t in range(L):
            sc = jnp.full((L,), scalev[t])
            zm = jnp.full((L,), zmxv[t])
            base = (n0 + t) * D
            for j in range(D // L):
                sj = s_acc[pl.ds(base + j * L, L)]
                mj = m_acc[pl.ds(base + j * L, L)]
                s_acc[pl.ds(base + j * L, L)] = sj * sc + zm * mj
        return carry
    lax.fori_loop(0, NPW // L, comb_group, 0)

    pltpu.sync_copy(s_acc.at[pl.ds(0, NPW * D)],
                    out_hbm.at[pl.ds(wid * (NPW * D), NPW * D)])


def kernel(z_agg_hard, edge_index, x):
    z = jnp.pad(z_agg_hard.reshape(3).astype(jnp.float32), (0, L - 3))
    src = edge_index[0].astype(jnp.int32)
    dst = edge_index[1].astype(jnp.int32)
    epack = lax.shift_left(src, 14) | dst
    u = lax.bitcast_convert_type(x.astype(jnp.bfloat16), jnp.uint16)
    k_idx = jnp.arange(DP)
    idx_lo = 32 * (k_idx // L) + (k_idx % L)
    w = (u[:, idx_lo].astype(jnp.uint32)
         | (u[:, idx_lo + L].astype(jnp.uint32) << 16)).astype(jnp.int32)
    out = _mp(z, epack, w.reshape(NH, D))
    return out.reshape(NPAD, D)[:N]


# speedup vs baseline: 2.6375x; 2.6375x over previous
import functools
import jax
import jax.numpy as jnp
from jax import lax
from jax.experimental import pallas as pl
from jax.experimental.pallas import tpu as pltpu
from jax.experimental.pallas import tpu_sc as plsc

N = 10000
E = 320000
D = 128
DP = D // 2
NH = N // 2
L = 16
NC = 2
NS = 16
NW = NC * NS
NPW = 320
NPAD = NW * NPW
C = 800
NCHUNK = E // C
FT = 10744
MBUF = FT + 2 * C + 64
NBIN = N + L
XR = 8
NXC = NH // XR
DUMP = NPW
NEGP = -8421505

_mesh = plsc.VectorSubcoreMesh(
    core_axis_name="c", subcore_axis_name="s", num_cores=NC, num_subcores=NS
)


@functools.partial(
    pl.kernel,
    out_type=jax.ShapeDtypeStruct((NPAD * D,), jnp.float32),
    mesh=_mesh,
    compiler_params=pltpu.CompilerParams(needs_layout_passes=False),
    scratch_types=[
        pltpu.VMEM(((NPW + 1) * D,), jnp.float32),
        pltpu.VMEM(((NPW + 1) * D,), jnp.float32),
        pltpu.VMEM((NPW + 2 * L,), jnp.float32),
        pltpu.VMEM((NBIN,), jnp.int32),
        pltpu.VMEM((MBUF,), jnp.int32),
        pltpu.VMEM((MBUF,), jnp.int32),
        pltpu.VMEM((C,), jnp.int32),
        pltpu.VMEM((C,), jnp.int32),
        pltpu.VMEM((XR, D), jnp.int32),
        pltpu.VMEM((XR, D), jnp.int32),
        pltpu.VMEM((L,), jnp.float32),
        pltpu.SemaphoreType.DMA,
        pltpu.SemaphoreType.DMA,
        pltpu.SemaphoreType.DMA,
        pltpu.SemaphoreType.DMA,
    ],
)
def _mp(z_hbm, ep_hbm, w_hbm, out_hbm,
        s_acc, m_acc, cnt_acc, bins, mpack, srt, chA, chB, xchA, xchB, zv,
        semA, semB, semXA, semXB):
    cid = lax.axis_index("c")
    sid = lax.axis_index("s")
    wid = cid * NS + sid
    lo = wid * NPW

    fzeros = jnp.zeros((L,), jnp.float32)
    fones = jnp.ones((L,), jnp.float32)
    izeros = jnp.zeros((L,), jnp.int32)
    iones = jnp.ones((L,), jnp.int32)
    negs = jnp.full((L,), -3.0e38, jnp.float32)
    dumpv = jnp.full((L,), DUMP << 14, jnp.int32)
    lowm = jnp.full((L,), 16383, jnp.int32)
    highm = jnp.full((L,), -65536, jnp.int32)
    sh16 = jnp.full((L,), 16, jnp.int32)
    lanes = lax.iota(jnp.int32, L)
    e1f = jnp.where(lanes == izeros, fones, fzeros)
    e1i = jnp.where(lanes == izeros, iones, izeros)
    lov14 = jnp.full((L,), lo * 16384, jnp.int32)
    hiv14 = jnp.full((L,), (lo + NPW) * 16384, jnp.int32)

    def init_acc(i, carry):
        s_acc[pl.ds(i * L, L)] = fzeros
        m_acc[pl.ds(i * L, L)] = negs
        return carry
    lax.fori_loop(0, (NPW + 1) * D // L, init_acc, 0)

    def init_cnt(i, carry):
        cnt_acc[pl.ds(i * L, L)] = fzeros
        return carry
    lax.fori_loop(0, (NPW + 2 * L) // L, init_cnt, 0)

    def init_bins(i, carry):
        bins[pl.ds(i * L, L)] = izeros
        return carry
    lax.fori_loop(0, NBIN // L, init_bins, 0)

    def init_srt(i, carry):
        srt[pl.ds(i * L, L)] = izeros
        return carry
    lax.fori_loop(0, MBUF // L, init_srt, 0)

    pltpu.sync_copy(z_hbm, zv)

    def issue_x(k, xb, sem):
        pltpu.async_copy(w_hbm.at[pl.ds(k * XR, XR), :], xb, sem)

    def wait_x(xb, sem):
        pltpu.make_async_copy(w_hbm.at[pl.ds(0, XR), :], xb, sem).wait()

    def walk(xi, xb):
        r0 = xi * (2 * XR)
        ks = bins[pl.ds(r0, L)][0]
        ke = bins[pl.ds(r0 + 2 * XR, L)][0]
        rh = r0 >> 1

        def edge(i, carry):
            lsp = srt[pl.ds(ks + i, L)][0]
            ls = lax.shift_right_logical(lsp, 14)
            dv = lsp & 16383
            baseS = ls * D
            sl = lax.shift_right_logical(dv, 1) - rh
            hof = (dv & 1) * DP
            for h in range(DP // L):
                wv = xb[sl, pl.ds(hof + h * L, L)]
                lof = plsc.bitcast(lax.shift_left(wv, sh16), jnp.float32)
                hif = plsc.bitcast(wv & highm, jnp.float32)
                plsc.addupdate(s_acc.at[pl.ds(baseS + h * 2 * L, L)], lof)
                plsc.addupdate(
                    s_acc.at[pl.ds(baseS + h * 2 * L + L, L)], hif)
                mlo = m_acc[pl.ds(baseS + h * 2 * L, L)]
                m_acc[pl.ds(baseS + h * 2 * L, L)] = jnp.maximum(mlo, lof)
                mhi = m_acc[pl.ds(baseS + h * 2 * L + L, L)]
                m_acc[pl.ds(baseS + h * 2 * L + L, L)] = (
                    jnp.maximum(mhi, hif))
            return carry
        lax.fori_loop(0, ke - ks, edge, 0)

    def flush(m):
        m16 = ((m + (L - 1)) >> 4) << 4
        mal = (m >> 4) << 4

        @pl.when(mal < m16)
        def _():
            v = mpack[pl.ds(mal, L)]
            posv = jnp.full((L,), mal, jnp.int32) + lanes
            mpack[pl.ds(mal, L)] = jnp.where(
                posv >= jnp.full((L,), m, jnp.int32), dumpv, v)

        nv = m16 >> 4

        issue_x(0, xchA, semXA)
        issue_x(1, xchB, semXB)

        def hist(i, carry):
            pv = mpack[pl.ds(i * L, L)]
            for t in range(L):
                dv = pv[t] & 16383
                plsc.addupdate(bins.at[pl.ds(dv, L)], e1i)
            return carry
        lax.fori_loop(0, nv, hist, 0)

        def pfx(i, tot):
            v = bins[pl.ds(i * L, L)]
            cum = plsc.cumsum(v)
            bins[pl.ds(i * L, L)] = cum + jnp.full((L,), tot, jnp.int32)
            return tot + cum[L - 1]
        lax.fori_loop(0, NBIN // L, pfx, 0)

        ne1i = izeros - e1i

        def place(i, carry):
            pv = mpack[pl.ds(i * L, L)]
            for t in range(L):
                lsp = pv[t]
                dv = lsp & 16383
                ls = lax.shift_right_logical(lsp, 14)
                p = bins[pl.ds(dv, L)][0] - 1
                plsc.addupdate(bins.at[pl.ds(dv, L)], ne1i)
                plsc.addupdate(cnt_acc.at[pl.ds(ls, L)], e1f)
                plsc.addupdate(
                    srt.at[pl.ds(p, L)],
                    jnp.where(lanes == izeros,
                              jnp.full((L,), lsp, jnp.int32), izeros))
            return carry
        lax.fori_loop(0, nv, place, 0)

        def xpair(cp, carry):
            ca = 2 * cp
            wait_x(xchA, semXA)
            walk(ca, xchA)

            @pl.when(ca + 2 < NXC)
            def _():
                issue_x(ca + 2, xchA, semXA)

            @pl.when(ca + 1 < NXC)
            def _():
                wait_x(xchB, semXB)
                walk(ca + 1, xchB)

                @pl.when(ca + 3 < NXC)
                def __():
                    issue_x(ca + 3, xchB, semXB)
            return carry
        lax.fori_loop(0, (NXC + 1) // 2, xpair, 0)

        def rz_bins(i, carry):
            bins[pl.ds(i * L, L)] = izeros
            return carry
        lax.fori_loop(0, NBIN // L, rz_bins, 0)

        def rz_srt(i, carry):
            srt[pl.ds(i * L, L)] = izeros
            return carry
        lax.fori_loop(0, nv, rz_srt, 0)

    def scan_chunk(cb, m0):
        def scan_body(i, off):
            ev = cb[pl.ds(i * L, L)]
            msk = (ev >= lov14) & (ev < hiv14)
            inc = jnp.where(msk, iones, izeros)
            pos = plsc.cumsum(inc)
            idx = jnp.full((L,), off - 1, jnp.int32) + pos
            plsc.store_scatter(mpack, [idx], ev - lov14, mask=msk)
            pc = plsc.all_reduce_population_count(msk)
            return off + pc[0]
        return lax.fori_loop(0, C // L, scan_body, m0)

    issue_chunk = lambda k, cb, sem: pltpu.async_copy(
        ep_hbm.at[pl.ds(k * C, C)], cb, sem)
    wait_chunk = lambda cb, sem: pltpu.make_async_copy(
        ep_hbm.at[pl.ds(0, C)], cb, sem).wait()

    issue_chunk(0, chA, semA)

    def chunk_pair(p, m):
        last = p == NCHUNK // 2

        k0 = 2 * p

        @pl.when(jnp.logical_not(last))
        def _():
            wait_chunk(chA, semA)
            issue_chunk(k0 + 1, chB, semB)
        m = jnp.where(last, m, scan_chunk(chA, m))

        @pl.when(jnp.logical_not(last))
        def _():
            wait_chunk(chB, semB)

            @pl.when(k0 + 2 < NCHUNK)
            def __():
                issue_chunk(k0 + 2, chA, semA)
        m = jnp.where(last, m, scan_chunk(chB, m))
        do_flush = (m >= FT) | (last & (m > 0))

        @pl.when(do_flush)
        def _():
            flush(m)
        return jnp.where(do_flush, 0, m)
    m = lax.fori_loop(0, NCHUNK // 2 + 1, chunk_pair, 0)

    zvec = zv[pl.ds(0, L)]
    z0v = jnp.full((L,), zvec[0])
    z1v = jnp.full((L,), zvec[1])
    z2v = jnp.full((L,), zvec[2])

    def comb_group(ng, carry):
        n0 = ng * L
        cv = cnt_acc[pl.ds(n0, L)]
        scalev = z0v + z1v / jnp.maximum(cv, fones)
        zmxv = jnp.where(cv > fzeros, z2v, fzeros)
        for t in range(L):
            sc = jnp.full((L,), scalev[t])
            zm = jnp.full((L,), zmxv[t])
            base = (n0 + t) * D
            for j in range(D // L):
                sj = s_acc[pl.ds(base + j * L, L)]
                mj = m_acc[pl.ds(base + j * L, L)]
                s_acc[pl.ds(base + j * L, L)] = sj * sc + zm * mj
        return carry
    lax.fori_loop(0, NPW // L, comb_group, 0)

    pltpu.sync_copy(s_acc.at[pl.ds(0, NPW * D)],
                    out_hbm.at[pl.ds(wid * (NPW * D), NPW * D)])


def kernel(z_agg_hard, edge_index, x):
    z = jnp.pad(z_agg_hard.reshape(3).astype(jnp.float32), (0, L - 3))
    src = edge_index[0].astype(jnp.int32)
    dst = edge_index[1].astype(jnp.int32)
    epack = lax.shift_left(src, 14) | dst
    u = lax.bitcast_convert_type(x.astype(jnp.bfloat16), jnp.uint16)
    k_idx = jnp.arange(DP)
    idx_lo = 32 * (k_idx // L) + (k_idx % L)
    w = (u[:, idx_lo].astype(jnp.uint32)
         | (u[:, idx_lo + L].astype(jnp.uint32) << 16)).astype(jnp.int32)
    out = _mp(z, epack, w.reshape(NH, D))
    return out.reshape(NPAD, D)[:N]
